# baseline (speedup 1.0000x reference)
import functools

import jax
import jax.numpy as jnp
from jax import lax
from jax.experimental import pallas as pl
from jax.experimental.pallas import tpu as pltpu
from jax.experimental.pallas import tpu_sc as plsc

_K = 512
_ALPHA = 0.01
_GAMMA = 1.0

_D = 32768
_B = 128
_NW = 32
_RPW = _B // _NW


def _sortable(x_f32):
    s = lax.bitcast_convert_type(x_f32, jnp.int32)
    return s ^ ((s >> 31) & jnp.int32(0x7FFFFFFF))


def _sortable_tc(x):
    s = lax.bitcast_convert_type(x, jnp.int32)
    return s ^ ((s >> 31) & jnp.int32(0x7FFFFFFF))


def _select_kth_key(rowbuf, cand, hist, lanes):
    ones = jnp.ones((16,), jnp.int32)
    zeros16 = jnp.zeros((16,), jnp.int32)
    k_rem = jnp.full((16,), _K, jnp.int32)
    prefix = zeros16
    n_cand = jnp.int32(_D)

    for level in range(4):
        shift = 24 - 8 * level
        flip = 0x80 if level == 0 else 0

        def clr(i, c):
            for u in range(8):
                hist[pl.ds(i * 128 + u * 16, 16)] = zeros16
            return c
        lax.fori_loop(0, 32, clr, 0)

        if level == 0:
            def build(i, c):
                for u in range(4):
                    key = _sortable(rowbuf[pl.ds((i * 4 + u) * 16, 16)])
                    digit = ((key >> shift) & 0xFF) ^ flip
                    plsc.addupdate_scatter(hist, [lanes * 256 + digit], ones)
                return c
            lax.fori_loop(0, _D // 64, build, 0)
        else:
            nv = (n_cand + 15) >> 4

            def build(i, c):
                key = cand[pl.ds(i * 16, 16)]
                valid = (i * 16 + lanes) < n_cand
                digit = ((key >> shift) & 0xFF) ^ flip
                plsc.addupdate_scatter(hist, [lanes * 256 + digit], ones,
                                       mask=valid)
                return c
            lax.fori_loop(0, nv, build, 0)

        def walk(i, st):
            carry, found, dwin, cntgt = st
            c = 15 - i
            tot = zeros16
            for s in range(16):
                tot = tot + hist[pl.ds(s * 4096 // 16 + c * 16, 16)]
            suf = lax.rev(jnp.cumsum(lax.rev(tot, (0,))), (0,))
            g = suf + carry
            in_mask = g >= k_rem
            cnt = plsc.all_reduce_population_count(in_mask)
            jstar = cnt - 1
            s_gt = jnp.sum(jnp.where(lanes > jstar, tot, 0))
            has = cnt > 0
            upd = has & jnp.logical_not(found)
            dwin = jnp.where(upd, c * 16 + jstar, dwin)
            cntgt = jnp.where(upd, carry + s_gt, cntgt)
            found = found | has
            carry = carry + jnp.sum(tot)
            return carry, found, dwin, cntgt

        init = (zeros16, jnp.zeros((16,), jnp.bool_), zeros16, zeros16)
        _, _, dwin, cntgt = lax.fori_loop(0, 16, walk, init)

        prefix = prefix | ((dwin ^ flip) << shift)
        k_rem = k_rem - cntgt

        if level < 3:
            if level == 0:
                def comp(i, off):
                    for u in range(4):
                        key = _sortable(rowbuf[pl.ds((i * 4 + u) * 16, 16)])
                        m = (((key >> shift) & 0xFF) ^ flip) == dwin
                        plsc.store_compressed(cand.at[pl.ds(off, 16)], key,
                                              mask=m)
                        off = off + jnp.sum(m.astype(jnp.int32))
                    return off
                n_cand = lax.fori_loop(0, _D // 64, comp, jnp.int32(0))
            else:
                nv = (n_cand + 15) >> 4

                def comp(i, off):
                    key = cand[pl.ds(i * 16, 16)]
                    valid = (i * 16 + lanes) < n_cand
                    m = valid & ((((key >> shift) & 0xFF) ^ flip) == dwin)
                    plsc.store_compressed(cand.at[pl.ds(off, 16)], key,
                                          mask=m)
                    return off + jnp.sum(m.astype(jnp.int32))
                n_cand = lax.fori_loop(0, nv, comp, jnp.int32(0))
    return prefix


def _thr_body(x_hbm, thr_hbm, rowbuf, cand, hist, thrv, sem):
    wid = lax.axis_index("s") * 2 + lax.axis_index("c")
    lanes = lax.iota(jnp.int32, 16)
    thr_acc = jnp.zeros((16,), jnp.int32)
    for r in range(_RPW):
        pltpu.async_copy(x_hbm.at[wid * _RPW + r], rowbuf, sem).wait()
        key = _select_kth_key(rowbuf, cand, hist, lanes)
        thr_acc = jnp.where(lanes == r, key, thr_acc)
    thrv[...] = thr_acc
    pltpu.sync_copy(thrv, thr_hbm.at[wid])


_thr_sc = functools.partial(
    pl.kernel,
    out_type=jax.ShapeDtypeStruct((_NW, 16), jnp.int32),
    mesh=plsc.VectorSubcoreMesh(core_axis_name="c", subcore_axis_name="s",
                                num_cores=2, num_subcores=16),
    compiler_params=pltpu.CompilerParams(needs_layout_passes=False),
    scratch_types=[
        pltpu.VMEM((_D,), jnp.float32),
        pltpu.VMEM((_D + 16,), jnp.int32),
        pltpu.VMEM((4096,), jnp.int32),
        pltpu.VMEM((16,), jnp.int32),
        pltpu.SemaphoreType.DMA,
    ],
)(_thr_body)


def _out_kernel(x_ref, thr_ref, duty_ref, out_ref):
    x = x_ref[...]
    mask = _sortable_tc(x) >= thr_ref[...]
    cc = jnp.sum(mask.astype(jnp.float32), axis=0, keepdims=True)
    duty_new = duty_ref[...] * (1.0 - _ALPHA) + (_ALPHA / x.shape[0]) * cc
    boost = jnp.exp(-_GAMMA * (duty_new - _K / _D))
    out_ref[...] = jnp.where(mask, x * boost, 0.0)


def kernel(x, duty):
    b, d = x.shape
    thr_packed = _thr_sc(x)
    thr = thr_packed[:, :_RPW].reshape(b, 1)

    cb = 2048
    out = pl.pallas_call(
        _out_kernel,
        grid=(d // cb,),
        in_specs=[
            pl.BlockSpec((b, cb), lambda j: (0, j)),
            pl.BlockSpec((b, 1), lambda j: (0, 0)),
            pl.BlockSpec((1, cb), lambda j: (0, j)),
        ],
        out_specs=pl.BlockSpec((b, cb), lambda j: (0, j)),
        out_shape=jax.ShapeDtypeStruct((b, d), jnp.float32),
    )(x, thr, duty)
    return out


# speedup vs baseline: 2.3905x; 2.3905x over previous
import functools

import jax
import jax.numpy as jnp
from jax import lax
from jax.experimental import pallas as pl
from jax.experimental.pallas import tpu as pltpu
from jax.experimental.pallas import tpu_sc as plsc

_K = 512
_ALPHA = 0.01
_GAMMA = 1.0

_D = 32768
_B = 128
_NW = 32
_RPW = _B // _NW


def _sortable(x_f32):
    s = lax.bitcast_convert_type(x_f32, jnp.int32)
    return s ^ ((s >> 31) & jnp.int32(0x7FFFFFFF))


def _sortable_tc(x):
    s = lax.bitcast_convert_type(x, jnp.int32)
    return s ^ ((s >> 31) & jnp.int32(0x7FFFFFFF))


def _select_kth_key(rowbuf, cand, hist, lanes):
    ones = jnp.ones((16,), jnp.int32)
    zeros16 = jnp.zeros((16,), jnp.int32)
    k_rem = jnp.full((16,), _K, jnp.int32)
    prefix = zeros16
    n_cand = jnp.int32(_D)

    for level in range(4):
        shift = 24 - 8 * level
        flip = 0x80 if level == 0 else 0

        @plsc.parallel_loop(0, 256, 1, unroll=8)
        def _(i):
            hist[pl.ds(i * 16, 16)] = zeros16

        if level == 0:
            @plsc.parallel_loop(0, _D // 16, 1, unroll=8)
            def _(i):
                key = _sortable(rowbuf[pl.ds(i * 16, 16)])
                digit = ((key >> shift) & 0xFF) ^ flip
                plsc.addupdate_scatter(hist, [lanes * 256 + digit], ones)
        else:
            nv = (n_cand + 15) >> 4
            n_cand_s = n_cand

            @plsc.parallel_loop(0, nv, 1, unroll=4)
            def _(i):
                key = cand[pl.ds(i * 16, 16)]
                valid = (i * 16 + lanes) < n_cand_s
                digit = ((key >> shift) & 0xFF) ^ flip
                plsc.addupdate_scatter(hist, [lanes * 256 + digit], ones,
                                       mask=valid)

        def walk(i, st):
            carry, found, dwin, cntgt = st
            c = 15 - i
            tot = zeros16
            for s in range(16):
                tot = tot + hist[pl.ds(s * 4096 // 16 + c * 16, 16)]
            suf = lax.rev(jnp.cumsum(lax.rev(tot, (0,))), (0,))
            g = suf + carry
            in_mask = g >= k_rem
            cnt = plsc.all_reduce_population_count(in_mask)
            jstar = cnt - 1
            s_gt = jnp.sum(jnp.where(lanes > jstar, tot, 0))
            has = cnt > 0
            upd = has & jnp.logical_not(found)
            dwin = jnp.where(upd, c * 16 + jstar, dwin)
            cntgt = jnp.where(upd, carry + s_gt, cntgt)
            found = found | has
            carry = carry + jnp.sum(tot)
            return carry, found, dwin, cntgt

        init = (zeros16, jnp.zeros((16,), jnp.bool_), zeros16, zeros16)
        _, _, dwin, cntgt = lax.fori_loop(0, 16, walk, init)

        prefix = prefix | ((dwin ^ flip) << shift)
        k_rem = k_rem - cntgt

        if level < 3:
            if level == 0:
                @plsc.parallel_loop(0, _D // 16, 1, unroll=8,
                                    carry=jnp.int32(0))
                def n_cand(i, off):
                    key = _sortable(rowbuf[pl.ds(i * 16, 16)])
                    m = (((key >> shift) & 0xFF) ^ flip) == dwin
                    plsc.store_compressed(cand.at[pl.ds(off, 16)], key,
                                          mask=m)
                    return off + jnp.sum(m.astype(jnp.int32))
            else:
                nv = (n_cand + 15) >> 4
                n_cand_s = n_cand

                @plsc.parallel_loop(0, nv, 1, unroll=4,
                                    carry=jnp.int32(0))
                def n_cand(i, off):
                    key = cand[pl.ds(i * 16, 16)]
                    valid = (i * 16 + lanes) < n_cand_s
                    m = valid & ((((key >> shift) & 0xFF) ^ flip) == dwin)
                    plsc.store_compressed(cand.at[pl.ds(off, 16)], key,
                                          mask=m)
                    return off + jnp.sum(m.astype(jnp.int32))
    return prefix


def _thr_body(x_hbm, thr_hbm, rowbuf, cand, hist, thrv, sem):
    wid = lax.axis_index("s") * 2 + lax.axis_index("c")
    lanes = lax.iota(jnp.int32, 16)
    thr_acc = jnp.zeros((16,), jnp.int32)
    for r in range(_RPW):
        pltpu.async_copy(x_hbm.at[wid * _RPW + r], rowbuf, sem).wait()
        key = _select_kth_key(rowbuf, cand, hist, lanes)
        thr_acc = jnp.where(lanes == r, key, thr_acc)
    thrv[...] = thr_acc
    pltpu.sync_copy(thrv, thr_hbm.at[wid])


_thr_sc = functools.partial(
    pl.kernel,
    out_type=jax.ShapeDtypeStruct((_NW, 16), jnp.int32),
    mesh=plsc.VectorSubcoreMesh(core_axis_name="c", subcore_axis_name="s",
                                num_cores=2, num_subcores=16),
    compiler_params=pltpu.CompilerParams(needs_layout_passes=False),
    scratch_types=[
        pltpu.VMEM((_D,), jnp.float32),
        pltpu.VMEM((_D + 16,), jnp.int32),
        pltpu.VMEM((4096,), jnp.int32),
        pltpu.VMEM((16,), jnp.int32),
        pltpu.SemaphoreType.DMA,
    ],
)(_thr_body)


def _out_kernel(x_ref, thr_ref, duty_ref, out_ref):
    x = x_ref[...]
    mask = _sortable_tc(x) >= thr_ref[...]
    cc = jnp.sum(mask.astype(jnp.float32), axis=0, keepdims=True)
    duty_new = duty_ref[...] * (1.0 - _ALPHA) + (_ALPHA / x.shape[0]) * cc
    boost = jnp.exp(-_GAMMA * (duty_new - _K / _D))
    out_ref[...] = jnp.where(mask, x * boost, 0.0)


def kernel(x, duty):
    b, d = x.shape
    thr_packed = _thr_sc(x)
    thr = thr_packed[:, :_RPW].reshape(b, 1)

    cb = 2048
    out = pl.pallas_call(
        _out_kernel,
        grid=(d // cb,),
        in_specs=[
            pl.BlockSpec((b, cb), lambda j: (0, j)),
            pl.BlockSpec((b, 1), lambda j: (0, 0)),
            pl.BlockSpec((1, cb), lambda j: (0, j)),
        ],
        out_specs=pl.BlockSpec((b, cb), lambda j: (0, j)),
        out_shape=jax.ShapeDtypeStruct((b, d), jnp.float32),
    )(x, thr, duty)
    return out


# speedup vs baseline: 2.5442x; 1.0643x over previous
import functools

import jax
import jax.numpy as jnp
from jax import lax
from jax.experimental import pallas as pl
from jax.experimental.pallas import tpu as pltpu
from jax.experimental.pallas import tpu_sc as plsc

_K = 512
_ALPHA = 0.01
_GAMMA = 1.0

_D = 32768
_B = 128
_NW = 32
_RPW = _B // _NW


def _sortable(x_f32):
    s = lax.bitcast_convert_type(x_f32, jnp.int32)
    return s ^ ((s >> 31) & jnp.int32(0x7FFFFFFF))


def _sortable_tc(x):
    s = lax.bitcast_convert_type(x, jnp.int32)
    return s ^ ((s >> 31) & jnp.int32(0x7FFFFFFF))


def _select_kth_key(rowbuf, cand, hist, lanes):
    ones = jnp.ones((16,), jnp.int32)
    zeros16 = jnp.zeros((16,), jnp.int32)
    k_rem = jnp.full((16,), _K, jnp.int32)
    prefix = zeros16
    n_cand = jnp.int32(_D)

    for level in range(4):
        shift = 24 - 8 * level
        flip = 0x80 if level == 0 else 0

        @plsc.parallel_loop(0, 256, 1, unroll=8)
        def _(i):
            hist[pl.ds(i * 16, 16)] = zeros16

        if level == 0:
            @plsc.parallel_loop(0, _D // 16, 1, unroll=8)
            def _(i):
                key = _sortable(rowbuf[pl.ds(i * 16, 16)])
                digit = ((key >> shift) & 0xFF) ^ flip
                plsc.addupdate_scatter(hist, [lanes * 256 + digit], ones)
        else:
            nv = (n_cand + 15) >> 4
            n_cand_s = n_cand

            @plsc.parallel_loop(0, nv, 1, unroll=4)
            def _(i):
                key = cand[pl.ds(i * 16, 16)]
                valid = (i * 16 + lanes) < n_cand_s
                digit = ((key >> shift) & 0xFF) ^ flip
                plsc.addupdate_scatter(hist, [lanes * 256 + digit], ones,
                                       mask=valid)

        def walk(i, st):
            carry, found, dwin, cntgt = st
            c = 15 - i
            tot = zeros16
            for s in range(16):
                tot = tot + hist[pl.ds(s * 4096 // 16 + c * 16, 16)]
            suf = lax.rev(jnp.cumsum(lax.rev(tot, (0,))), (0,))
            g = suf + carry
            in_mask = g >= k_rem
            cnt = plsc.all_reduce_population_count(in_mask)
            jstar = cnt - 1
            s_gt = jnp.sum(jnp.where(lanes > jstar, tot, 0))
            has = cnt > 0
            upd = has & jnp.logical_not(found)
            dwin = jnp.where(upd, c * 16 + jstar, dwin)
            cntgt = jnp.where(upd, carry + s_gt, cntgt)
            found = found | has
            carry = carry + jnp.sum(tot)
            return carry, found, dwin, cntgt

        init = (zeros16, jnp.zeros((16,), jnp.bool_), zeros16, zeros16)
        _, _, dwin, cntgt = lax.fori_loop(0, 16, walk, init)

        prefix = prefix | ((dwin ^ flip) << shift)
        k_rem = k_rem - cntgt

        if level < 3:
            if level == 0:
                @plsc.parallel_loop(0, _D // 16, 1, unroll=8,
                                    carry=jnp.int32(0))
                def n_cand(i, off):
                    key = _sortable(rowbuf[pl.ds(i * 16, 16)])
                    m = (((key >> shift) & 0xFF) ^ flip) == dwin
                    plsc.store_compressed(cand.at[pl.ds(off, 16)], key,
                                          mask=m)
                    return off + jnp.sum(m.astype(jnp.int32))
            else:
                nv = (n_cand + 15) >> 4
                n_cand_s = n_cand

                @plsc.parallel_loop(0, nv, 1, unroll=4,
                                    carry=jnp.int32(0))
                def n_cand(i, off):
                    key = cand[pl.ds(i * 16, 16)]
                    valid = (i * 16 + lanes) < n_cand_s
                    m = valid & ((((key >> shift) & 0xFF) ^ flip) == dwin)
                    plsc.store_compressed(cand.at[pl.ds(off, 16)], key,
                                          mask=m)
                    return off + jnp.sum(m.astype(jnp.int32))
    return prefix


def _thr_body(x_hbm, thr_hbm, rowbuf, cand, hist, thrv, sem):
    wid = lax.axis_index("s") * 2 + lax.axis_index("c")
    lanes = lax.iota(jnp.int32, 16)
    thr_acc = jnp.zeros((16,), jnp.int32)
    for r in range(_RPW):
        pltpu.async_copy(x_hbm.at[wid * _RPW + r], rowbuf, sem).wait()
        key = _select_kth_key(rowbuf, cand, hist, lanes)
        thr_acc = jnp.where(lanes == r, key, thr_acc)
    thrv[...] = thr_acc
    pltpu.sync_copy(thrv, thr_hbm.at[wid])


_thr_sc = functools.partial(
    pl.kernel,
    out_type=jax.ShapeDtypeStruct((_NW, 16), jnp.int32),
    mesh=plsc.VectorSubcoreMesh(core_axis_name="c", subcore_axis_name="s",
                                num_cores=2, num_subcores=16),
    compiler_params=pltpu.CompilerParams(needs_layout_passes=False),
    scratch_types=[
        pltpu.VMEM((_D,), jnp.float32),
        pltpu.VMEM((_D + 16,), jnp.int32),
        pltpu.VMEM((4096,), jnp.int32),
        pltpu.VMEM((16,), jnp.int32),
        pltpu.SemaphoreType.DMA,
    ],
)(_thr_body)


def _out_kernel(x_ref, thr_ref, duty_ref, out_ref):
    x = x_ref[...]
    t = thr_ref[...]
    thr_f = lax.bitcast_convert_type(
        t ^ ((t >> 31) & jnp.int32(0x7FFFFFFF)), jnp.float32)
    mask = x >= thr_f
    cc = jnp.sum(mask.astype(jnp.float32), axis=0, keepdims=True)
    duty_new = duty_ref[...] * (1.0 - _ALPHA) + (_ALPHA / x.shape[0]) * cc
    boost = jnp.exp(-_GAMMA * (duty_new - _K / _D))
    out_ref[...] = jnp.where(mask, x * boost, 0.0)


def kernel(x, duty):
    b, d = x.shape
    thr_packed = _thr_sc(x)
    thr = thr_packed[:, :_RPW].reshape(b, 1)

    cb = 8192
    out = pl.pallas_call(
        _out_kernel,
        grid=(d // cb,),
        in_specs=[
            pl.BlockSpec((b, cb), lambda j: (0, j)),
            pl.BlockSpec((b, 1), lambda j: (0, 0)),
            pl.BlockSpec((1, cb), lambda j: (0, j)),
        ],
        out_specs=pl.BlockSpec((b, cb), lambda j: (0, j)),
        out_shape=jax.ShapeDtypeStruct((b, d), jnp.float32),
    )(x, thr, duty)
    return out
